# baseline (speedup 1.0000x reference)
import jax
import jax.numpy as jnp
from jax import lax
from jax.experimental import pallas as pl
from jax.experimental.pallas import tpu as pltpu
from jax.experimental.pallas import tpu_sc as plsc

NC = 2
NS = 16
LN = 16
C = 12800
K = 80
BE = 2000


def _linear_body(x_ref, wt_ref, b_ref, o_ref):
    o_ref[...] = (
        jnp.dot(x_ref[...], wt_ref[...], preferred_element_type=jnp.float32)
        + b_ref[...]
    )


def _linear(x, W, b, R=2000):
    N, D = x.shape
    return pl.pallas_call(
        _linear_body,
        grid=(N // R,),
        in_specs=[
            pl.BlockSpec((R, D), lambda i: (i, 0)),
            pl.BlockSpec((D, D), lambda i: (0, 0)),
            pl.BlockSpec((1, D), lambda i: (0, 0)),
        ],
        out_specs=pl.BlockSpec((R, D), lambda i: (i, 0)),
        out_shape=jax.ShapeDtypeStruct((N, D), jnp.float32),
    )(x, W.T, b.reshape(1, D))


def _update_body(nparts, x_ref, wt_ref, bs_ref, fm_ref, g_ref, bb_ref, *rest):
    aggs = rest[:nparts]
    degs = rest[nparts:2 * nparts]
    o_ref = rest[-1]
    x = x_ref[...]
    agg = aggs[0][...]
    for r in aggs[1:]:
        agg = agg + r[...]
    deg = degs[0][...]
    for r in degs[1:]:
        deg = deg + r[...]
    fm = fm_ref[...]
    d = jnp.maximum(deg, 1.0)
    a = (agg / d) * fm
    cc = (
        jnp.dot(x, wt_ref[...], preferred_element_type=jnp.float32)
        + bs_ref[...]
        + a
    )
    mu = jnp.mean(cc, axis=-1, keepdims=True)
    var = jnp.mean((cc - mu) ** 2, axis=-1, keepdims=True)
    cc = (cc - mu) * lax.rsqrt(var + 1e-5) * g_ref[...] + bb_ref[...]
    cc = cc + x
    cc = 1.0507009873554805 * jnp.where(
        cc > 0,
        cc,
        1.6732632423543772 * (jnp.exp(jnp.minimum(cc, 0.0)) - 1.0),
    )
    o_ref[...] = jnp.where(fm > 0.5, cc, x)


def _update(x, agg_parts, deg_parts, fm, Ws, bs, g, bb, R=2000):
    import functools
    N, D = x.shape
    nparts = len(agg_parts)
    row_spec = pl.BlockSpec((R, D), lambda i: (i, 0))
    col_spec = pl.BlockSpec((1, D), lambda i: (0, 0))
    deg_spec = pl.BlockSpec((R, 1), lambda i: (i, 0))
    return pl.pallas_call(
        functools.partial(_update_body, nparts),
        grid=(N // R,),
        in_specs=[
            row_spec,
            pl.BlockSpec((D, D), lambda i: (0, 0)),
            col_spec,
            col_spec,
            col_spec,
            col_spec,
        ]
        + [row_spec] * nparts
        + [deg_spec] * nparts,
        out_specs=row_spec,
        out_shape=jax.ShapeDtypeStruct((N, D), jnp.float32),
    )(
        x,
        Ws.T,
        bs.reshape(1, D),
        fm.reshape(1, D),
        g.reshape(1, D),
        bb.reshape(1, D),
        *[a for a in agg_parts],
        *[d.reshape(-1, 1) for d in deg_parts],
    )


def _sc_agg(h_a, h_b, src_p2c, dst_p2c, um1, src_c2p, dst_c2p, um2):
    E = src_p2c.shape[0]
    D = h_a.shape[1]
    n_chunks_b = 4

    def body(h_a_h, h_b_h, s1_h, d1_h, u1_h, s2_h, d2_h, u2_h,
             z2d_h, z1d_h, ones_h,
             agg_b_o, deg_b_o, agg_a0_o, deg_a0_o, agg_a1_o, deg_a1_o,
             src_v, dst_v, um_v, gidx, didx, gsm, dsm, rows_v, ones_v,
             zdeg_v, agg_sh, deg_sh, sem):
        c = lax.axis_index("c")
        s = lax.axis_index("s")
        rpt = C // NS
        row0 = s * rpt
        pltpu.sync_copy(ones_h, ones_v)

        def run_pass(src_h, dst_h, um_h, h_h, ebase, nblocks, lo):
            pltpu.sync_copy(z2d_h.at[pl.ds(0, rpt)],
                            agg_sh.at[pl.ds(row0, rpt)])
            pltpu.sync_copy(z1d_h.at[pl.ds(0, rpt)], zdeg_v)
            pltpu.sync_copy(zdeg_v, deg_sh.at[pl.ds(row0, rpt)])
            plsc.subcore_barrier()
            hi = lo + C

            def blk(b, _):
                eoff = ebase + b * BE
                pltpu.sync_copy(src_h.at[pl.ds(eoff, BE)], src_v)
                pltpu.sync_copy(dst_h.at[pl.ds(eoff, BE)], dst_v)
                pltpu.sync_copy(um_h.at[pl.ds(eoff, BE)], um_v)

                def vec(i, _):
                    sl = pl.ds(i * LN, LN)
                    s16 = src_v[sl]
                    d16 = dst_v[sl]
                    u16 = um_v[sl]
                    m = (d16 >= lo) & (d16 < hi) & (u16 > 0.5)
                    gidx[sl] = jnp.where(m, s16, 0)
                    didx[sl] = jnp.where(m, d16 - lo, C)
                    return 0

                lax.fori_loop(0, BE // LN, vec, 0)

                def drain(j, _):
                    o = j * K
                    for t in range(K // LN):
                        gsm[pl.ds(t * LN, LN)] = gidx[pl.ds(o + t * LN, LN)]
                        dsm[pl.ds(t * LN, LN)] = didx[pl.ds(o + t * LN, LN)]
                    pltpu.async_copy(h_h.at[gsm], rows_v, sem).wait()
                    pltpu.sync_copy(rows_v, agg_sh.at[dsm], add=True)
                    pltpu.sync_copy(ones_v, deg_sh.at[dsm], add=True)
                    return 0

                lax.fori_loop(0, BE // K, drain, 0)
                return 0

            lax.fori_loop(0, nblocks, blk, 0)
            plsc.subcore_barrier()

        epw = E // NS
        for p in range(2):
            lo = (2 * c + p) * C
            run_pass(s1_h, d1_h, u1_h, h_a_h, s * epw, E // NS // BE, lo)
            pltpu.sync_copy(agg_sh.at[pl.ds(row0, rpt)],
                            agg_b_o.at[pl.ds(lo + row0, rpt)])
            pltpu.sync_copy(deg_sh.at[pl.ds(row0, rpt)], zdeg_v)
            pltpu.sync_copy(zdeg_v, deg_b_o.at[pl.ds(lo + row0, rpt)])

        eh = E // 2
        run_pass(s2_h, d2_h, u2_h, h_b_h, c * eh + s * (eh // NS),
                 eh // NS // BE, 0)

        @pl.when(c == 0)
        def _():
            pltpu.sync_copy(agg_sh.at[pl.ds(row0, rpt)],
                            agg_a0_o.at[pl.ds(row0, rpt)])
            pltpu.sync_copy(deg_sh.at[pl.ds(row0, rpt)], zdeg_v)
            pltpu.sync_copy(zdeg_v, deg_a0_o.at[pl.ds(row0, rpt)])

        @pl.when(c == 1)
        def _():
            pltpu.sync_copy(agg_sh.at[pl.ds(row0, rpt)],
                            agg_a1_o.at[pl.ds(row0, rpt)])
            pltpu.sync_copy(deg_sh.at[pl.ds(row0, rpt)], zdeg_v)
            pltpu.sync_copy(zdeg_v, deg_a1_o.at[pl.ds(row0, rpt)])

    f32 = jnp.float32
    i32 = jnp.int32
    out_type = [
        jax.ShapeDtypeStruct((n_chunks_b * C, D), f32),
        jax.ShapeDtypeStruct((n_chunks_b * C,), f32),
        jax.ShapeDtypeStruct((C, D), f32),
        jax.ShapeDtypeStruct((C,), f32),
        jax.ShapeDtypeStruct((C, D), f32),
        jax.ShapeDtypeStruct((C,), f32),
    ]
    scratch_types = [
        pltpu.VMEM((BE,), i32),
        pltpu.VMEM((BE,), i32),
        pltpu.VMEM((BE,), f32),
        pltpu.VMEM((BE,), i32),
        pltpu.VMEM((BE,), i32),
        pltpu.VMEM((K,), i32),
        pltpu.VMEM((K,), i32),
        pltpu.VMEM((K, D), f32),
        pltpu.VMEM((K,), f32),
        pltpu.VMEM((C // NS,), f32),
        pltpu.VMEM_SHARED((C + 8, D), f32),
        pltpu.VMEM_SHARED((C + 8,), f32),
        pltpu.SemaphoreType.DMA,
    ]
    z2d = jnp.zeros((C // NS, D), f32)
    z1d = jnp.zeros((C // NS,), f32)
    ones = jnp.ones((K,), f32)
    fn = pl.kernel(
        body,
        out_type=out_type,
        mesh=plsc.VectorSubcoreMesh(core_axis_name="c", subcore_axis_name="s"),
        scratch_types=scratch_types,
    )
    return fn(h_a, h_b, src_p2c, dst_p2c, um1, src_c2p, dst_c2p, um2,
              z2d, z1d, ones)


def kernel(x_a, x_b, src_p2c, dst_p2c, src_c2p, dst_c2p, um_p2c, um_c2p,
           fm_a, fm_b, W_p2c, b_p2c, W_c2p, b_c2p, W_self_a, b_self_a,
           W_self_b, b_self_b, ln_g_a, ln_b_a, ln_g_b, ln_b_b):
    N_A = x_a.shape[0]
    N_B = x_b.shape[0]
    um1 = um_p2c.reshape(-1)
    um2 = um_c2p.reshape(-1)
    h_a = _linear(x_a, W_p2c, b_p2c)
    h_b = _linear(x_b, W_c2p, b_c2p)
    agg_b, deg_b, agg_a0, deg_a0, agg_a1, deg_a1 = _sc_agg(
        h_a, h_b, src_p2c, dst_p2c, um1, src_c2p, dst_c2p, um2)
    out_a = _update(
        x_a,
        [agg_a0[:N_A], agg_a1[:N_A]],
        [deg_a0[:N_A], deg_a1[:N_A]],
        fm_a, W_self_a, b_self_a, ln_g_a, ln_b_a,
    )
    out_b = _update(
        x_b,
        [agg_b[:N_B]],
        [deg_b[:N_B]],
        fm_b, W_self_b, b_self_b, ln_g_b, ln_b_b,
    )
    return (out_a, out_b)


# speedup vs baseline: 1.0010x; 1.0010x over previous
import jax
import jax.numpy as jnp
from jax import lax
from jax.experimental import pallas as pl
from jax.experimental.pallas import tpu as pltpu
from jax.experimental.pallas import tpu_sc as plsc

NC = 2
NS = 16
LN = 16
C = 12800
K = 80
BE = 2000


def _linear_body(x_ref, wt_ref, b_ref, o_ref):
    o_ref[...] = (
        jnp.dot(x_ref[...], wt_ref[...], preferred_element_type=jnp.float32)
        + b_ref[...]
    )


def _linear(x, W, b, R=2000):
    N, D = x.shape
    return pl.pallas_call(
        _linear_body,
        grid=(N // R,),
        in_specs=[
            pl.BlockSpec((R, D), lambda i: (i, 0)),
            pl.BlockSpec((D, D), lambda i: (0, 0)),
            pl.BlockSpec((1, D), lambda i: (0, 0)),
        ],
        out_specs=pl.BlockSpec((R, D), lambda i: (i, 0)),
        out_shape=jax.ShapeDtypeStruct((N, D), jnp.float32),
    )(x, W.T, b.reshape(1, D))


def _update_body(nparts, x_ref, wt_ref, bs_ref, fm_ref, g_ref, bb_ref, *rest):
    aggs = rest[:nparts]
    degs = rest[nparts:2 * nparts]
    o_ref = rest[-1]
    x = x_ref[...]
    agg = aggs[0][...]
    for r in aggs[1:]:
        agg = agg + r[...]
    deg = degs[0][...]
    for r in degs[1:]:
        deg = deg + r[...]
    fm = fm_ref[...]
    d = jnp.maximum(deg, 1.0)
    a = (agg / d) * fm
    cc = (
        jnp.dot(x, wt_ref[...], preferred_element_type=jnp.float32)
        + bs_ref[...]
        + a
    )
    mu = jnp.mean(cc, axis=-1, keepdims=True)
    var = jnp.mean((cc - mu) ** 2, axis=-1, keepdims=True)
    cc = (cc - mu) * lax.rsqrt(var + 1e-5) * g_ref[...] + bb_ref[...]
    cc = cc + x
    cc = 1.0507009873554805 * jnp.where(
        cc > 0,
        cc,
        1.6732632423543772 * (jnp.exp(jnp.minimum(cc, 0.0)) - 1.0),
    )
    o_ref[...] = jnp.where(fm > 0.5, cc, x)


def _update(x, agg_parts, deg_parts, fm, Ws, bs, g, bb, R=2000):
    import functools
    N, D = x.shape
    nparts = len(agg_parts)
    row_spec = pl.BlockSpec((R, D), lambda i: (i, 0))
    col_spec = pl.BlockSpec((1, D), lambda i: (0, 0))
    deg_spec = pl.BlockSpec((R, 1), lambda i: (i, 0))
    return pl.pallas_call(
        functools.partial(_update_body, nparts),
        grid=(N // R,),
        in_specs=[
            row_spec,
            pl.BlockSpec((D, D), lambda i: (0, 0)),
            col_spec,
            col_spec,
            col_spec,
            col_spec,
        ]
        + [row_spec] * nparts
        + [deg_spec] * nparts,
        out_specs=row_spec,
        out_shape=jax.ShapeDtypeStruct((N, D), jnp.float32),
    )(
        x,
        Ws.T,
        bs.reshape(1, D),
        fm.reshape(1, D),
        g.reshape(1, D),
        bb.reshape(1, D),
        *[a for a in agg_parts],
        *[d.reshape(-1, 1) for d in deg_parts],
    )


def _sc_agg(h_a, h_b, src_p2c, dst_p2c, um1, src_c2p, dst_c2p, um2):
    E = src_p2c.shape[0]
    D = h_a.shape[1]
    n_chunks_b = 4

    def body(h_a_h, h_b_h, s1_h, d1_h, u1_h, s2_h, d2_h, u2_h,
             z2d_h, z1d_h, ones_h,
             agg_b_o, deg_b_o, agg_a0_o, deg_a0_o, agg_a1_o, deg_a1_o,
             src_v, dst_v, um_v, gidx, didx, gsm, dsm, rows_v, ones_v,
             zdeg_v, agg_sh, deg_sh, sem):
        c = lax.axis_index("c")
        s = lax.axis_index("s")
        rpt = C // NS
        row0 = s * rpt
        pltpu.sync_copy(ones_h, ones_v)

        def run_pass(src_h, dst_h, um_h, h_h, ebase, nblocks, lo):
            pltpu.sync_copy(z2d_h.at[pl.ds(0, rpt)],
                            agg_sh.at[pl.ds(row0, rpt)])
            pltpu.sync_copy(z1d_h.at[pl.ds(0, rpt)], zdeg_v)
            pltpu.sync_copy(zdeg_v, deg_sh.at[pl.ds(row0, rpt)])
            plsc.subcore_barrier()
            hi = lo + C

            def blk(b, _):
                eoff = ebase + b * BE
                pltpu.sync_copy(src_h.at[pl.ds(eoff, BE)], src_v)
                pltpu.sync_copy(dst_h.at[pl.ds(eoff, BE)], dst_v)
                pltpu.sync_copy(um_h.at[pl.ds(eoff, BE)], um_v)

                def vec(i, _):
                    sl = pl.ds(i * LN, LN)
                    s16 = src_v[sl]
                    d16 = dst_v[sl]
                    u16 = um_v[sl]
                    m = (d16 >= lo) & (d16 < hi) & (u16 > 0.5)
                    gidx[sl] = jnp.where(m, s16, 0)
                    didx[sl] = jnp.where(m, d16 - lo, C + (s16 & 511))
                    return 0

                lax.fori_loop(0, BE // LN, vec, 0)

                def drain(j, _):
                    o = j * K
                    for t in range(K // LN):
                        gsm[pl.ds(t * LN, LN)] = gidx[pl.ds(o + t * LN, LN)]
                        dsm[pl.ds(t * LN, LN)] = didx[pl.ds(o + t * LN, LN)]
                    pltpu.async_copy(h_h.at[gsm], rows_v, sem).wait()
                    pltpu.sync_copy(rows_v, agg_sh.at[dsm], add=True)
                    pltpu.sync_copy(ones_v, deg_sh.at[dsm], add=True)
                    return 0

                lax.fori_loop(0, BE // K, drain, 0)
                return 0

            lax.fori_loop(0, nblocks, blk, 0)
            plsc.subcore_barrier()

        epw = E // NS
        for p in range(2):
            lo = (2 * c + p) * C
            run_pass(s1_h, d1_h, u1_h, h_a_h, s * epw, E // NS // BE, lo)
            pltpu.sync_copy(agg_sh.at[pl.ds(row0, rpt)],
                            agg_b_o.at[pl.ds(lo + row0, rpt)])
            pltpu.sync_copy(deg_sh.at[pl.ds(row0, rpt)], zdeg_v)
            pltpu.sync_copy(zdeg_v, deg_b_o.at[pl.ds(lo + row0, rpt)])

        eh = E // 2
        run_pass(s2_h, d2_h, u2_h, h_b_h, c * eh + s * (eh // NS),
                 eh // NS // BE, 0)

        @pl.when(c == 0)
        def _():
            pltpu.sync_copy(agg_sh.at[pl.ds(row0, rpt)],
                            agg_a0_o.at[pl.ds(row0, rpt)])
            pltpu.sync_copy(deg_sh.at[pl.ds(row0, rpt)], zdeg_v)
            pltpu.sync_copy(zdeg_v, deg_a0_o.at[pl.ds(row0, rpt)])

        @pl.when(c == 1)
        def _():
            pltpu.sync_copy(agg_sh.at[pl.ds(row0, rpt)],
                            agg_a1_o.at[pl.ds(row0, rpt)])
            pltpu.sync_copy(deg_sh.at[pl.ds(row0, rpt)], zdeg_v)
            pltpu.sync_copy(zdeg_v, deg_a1_o.at[pl.ds(row0, rpt)])

    f32 = jnp.float32
    i32 = jnp.int32
    out_type = [
        jax.ShapeDtypeStruct((n_chunks_b * C, D), f32),
        jax.ShapeDtypeStruct((n_chunks_b * C,), f32),
        jax.ShapeDtypeStruct((C, D), f32),
        jax.ShapeDtypeStruct((C,), f32),
        jax.ShapeDtypeStruct((C, D), f32),
        jax.ShapeDtypeStruct((C,), f32),
    ]
    scratch_types = [
        pltpu.VMEM((BE,), i32),
        pltpu.VMEM((BE,), i32),
        pltpu.VMEM((BE,), f32),
        pltpu.VMEM((BE,), i32),
        pltpu.VMEM((BE,), i32),
        pltpu.VMEM((K,), i32),
        pltpu.VMEM((K,), i32),
        pltpu.VMEM((K, D), f32),
        pltpu.VMEM((K,), f32),
        pltpu.VMEM((C // NS,), f32),
        pltpu.VMEM_SHARED((C + 512, D), f32),
        pltpu.VMEM_SHARED((C + 512,), f32),
        pltpu.SemaphoreType.DMA,
    ]
    z2d = jnp.zeros((C // NS, D), f32)
    z1d = jnp.zeros((C // NS,), f32)
    ones = jnp.ones((K,), f32)
    fn = pl.kernel(
        body,
        out_type=out_type,
        mesh=plsc.VectorSubcoreMesh(core_axis_name="c", subcore_axis_name="s"),
        scratch_types=scratch_types,
    )
    return fn(h_a, h_b, src_p2c, dst_p2c, um1, src_c2p, dst_c2p, um2,
              z2d, z1d, ones)


def kernel(x_a, x_b, src_p2c, dst_p2c, src_c2p, dst_c2p, um_p2c, um_c2p,
           fm_a, fm_b, W_p2c, b_p2c, W_c2p, b_c2p, W_self_a, b_self_a,
           W_self_b, b_self_b, ln_g_a, ln_b_a, ln_g_b, ln_b_b):
    N_A = x_a.shape[0]
    N_B = x_b.shape[0]
    um1 = um_p2c.reshape(-1)
    um2 = um_c2p.reshape(-1)
    h_a = _linear(x_a, W_p2c, b_p2c)
    h_b = _linear(x_b, W_c2p, b_c2p)
    agg_b, deg_b, agg_a0, deg_a0, agg_a1, deg_a1 = _sc_agg(
        h_a, h_b, src_p2c, dst_p2c, um1, src_c2p, dst_c2p, um2)
    out_a = _update(
        x_a,
        [agg_a0[:N_A], agg_a1[:N_A]],
        [deg_a0[:N_A], deg_a1[:N_A]],
        fm_a, W_self_a, b_self_a, ln_g_a, ln_b_a,
    )
    out_b = _update(
        x_b,
        [agg_b[:N_B]],
        [deg_b[:N_B]],
        fm_b, W_self_b, b_self_b, ln_g_b, ln_b_b,
    )
    return (out_a, out_b)
